# Initial kernel scaffold; baseline (speedup 1.0000x reference)
import jax
import jax.numpy as jnp
from jax.experimental import pallas as pl


def kernel(x, seg, word_emb, pos_emb, tok_emb, gamma, beta):
    raise NotImplementedError("write your pallas kernel here")


# speedup vs baseline: 3.1128x; 3.1128x over previous
import functools

import jax
import jax.numpy as jnp
from jax import lax
from jax.experimental import pallas as pl
from jax.experimental.pallas import tpu as pltpu
from jax.experimental.pallas import tpu_sc as plsc

DIM = 64
EPS = 1e-12

NC = 2
NS = 16
NW = NC * NS

CHUNK = 128


def _sc_gather_body(nper, nchunks, idx_hbm, tab_hbm, out_hbm, idx_v, rows_v, sem):
    wid = lax.axis_index("s") * NC + lax.axis_index("c")
    base_w = wid * nper

    def step(i, carry):
        base = base_w + i * CHUNK
        pltpu.sync_copy(idx_hbm.at[pl.ds(base, CHUNK)], idx_v)
        pltpu.async_copy(tab_hbm.at[idx_v], rows_v, sem).wait()
        pltpu.sync_copy(rows_v, out_hbm.at[pl.ds(base, CHUNK)])
        return carry

    lax.fori_loop(0, nchunks, step, 0)


def _sc_gather(word_emb, idx_flat):
    n = idx_flat.shape[0]
    assert n % (NW * CHUNK) == 0
    nper = n // NW
    nchunks = nper // CHUNK
    mesh = plsc.VectorSubcoreMesh(core_axis_name="c", subcore_axis_name="s")
    return pl.kernel(
        functools.partial(_sc_gather_body, nper, nchunks),
        out_type=jax.ShapeDtypeStruct((n, DIM), jnp.float32),
        mesh=mesh,
        scratch_types=[
            pltpu.VMEM((CHUNK,), jnp.int32),
            pltpu.VMEM((CHUNK, DIM), jnp.float32),
            pltpu.SemaphoreType.DMA,
        ],
        compiler_params=pltpu.CompilerParams(use_tc_tiling_on_sc=False),
    )(idx_flat, word_emb)


def _tc_ln_body(w_ref, seg_ref, pos_ref, tok_ref, gam_ref, bet_ref, o_ref):
    w = w_ref[...]
    seg = seg_ref[...]
    pos = pos_ref[...]
    tok0 = tok_ref[0]
    tok1 = tok_ref[1]
    t = jnp.where(seg == 0, tok0[None, None, :], tok1[None, None, :])
    e = w + pos[None] + t
    m = jnp.mean(e, axis=-1, keepdims=True)
    d = e - m
    var = jnp.mean(d * d, axis=-1, keepdims=True)
    o_ref[...] = (d * lax.rsqrt(var + EPS)) * gam_ref[0][None, None, :] + bet_ref[0][None, None, :]


def _tc_ln(w, seg, pos, tok, gamma, beta):
    b, l, d = w.shape
    bb = 32
    grid = (b // bb,)
    return pl.pallas_call(
        _tc_ln_body,
        grid=grid,
        in_specs=[
            pl.BlockSpec((bb, l, d), lambda i: (i, 0, 0)),
            pl.BlockSpec((bb, l, 1), lambda i: (i, 0, 0)),
            pl.BlockSpec((l, d), lambda i: (0, 0)),
            pl.BlockSpec((2, d), lambda i: (0, 0)),
            pl.BlockSpec((1, d), lambda i: (0, 0)),
            pl.BlockSpec((1, d), lambda i: (0, 0)),
        ],
        out_specs=pl.BlockSpec((bb, l, d), lambda i: (i, 0, 0)),
        out_shape=jax.ShapeDtypeStruct((b, l, d), jnp.float32),
    )(w, seg, pos, tok, gamma, beta)


def kernel(x, seg, word_emb, pos_emb, tok_emb, gamma, beta):
    b, l = x.shape
    idx_flat = x.reshape(-1).astype(jnp.int32)
    w = _sc_gather(word_emb, idx_flat)
    w = w.reshape(b, l, DIM)
    return _tc_ln(
        w,
        seg.reshape(b, l, 1),
        pos_emb[:l],
        tok_emb,
        gamma.reshape(1, DIM),
        beta.reshape(1, DIM),
    )


# speedup vs baseline: 3.3079x; 1.0627x over previous
import functools

import jax
import jax.numpy as jnp
from jax import lax
from jax.experimental import pallas as pl
from jax.experimental.pallas import tpu as pltpu
from jax.experimental.pallas import tpu_sc as plsc

DIM = 64
EPS = 1e-12

NC = 2
NS = 16
NW = NC * NS

CHUNK = 128
RING = 8


def _sc_gather_body(nchunks, idx_hbm, tab_hbm, out_hbm, idx_v, rows_v, gsem, wsem):
    wid = lax.axis_index("s") * NC + lax.axis_index("c")
    chunk0 = wid * nchunks

    pltpu.sync_copy(idx_hbm.at[pl.ds(chunk0, nchunks)], idx_v)

    def g_start(c, r):
        pltpu.async_copy(tab_hbm.at[idx_v.at[c]], rows_v.at[r], gsem.at[r])

    def g_wait(c, r):
        pltpu.make_async_copy(tab_hbm.at[idx_v.at[c]], rows_v.at[r], gsem.at[r]).wait()

    def w_start(c, r):
        pltpu.async_copy(rows_v.at[r], out_hbm.at[pl.ds((chunk0 + c) * CHUNK, CHUNK)], wsem.at[r])

    def w_wait(c, r):
        pltpu.make_async_copy(rows_v.at[r], out_hbm.at[pl.ds((chunk0 + c) * CHUNK, CHUNK)], wsem.at[r]).wait()

    ngroups = nchunks // RING
    for r in range(RING):
        g_start(r, r)

    def group(g, carry):
        base = g * RING
        for r in range(RING):
            g_wait(base + r, r)
            w_start(base + r, r)
        for r in range(RING):
            w_wait(base + r, r)
            g_start(base + RING + r, r)
        return carry

    lax.fori_loop(0, ngroups - 1, group, 0)

    base = (ngroups - 1) * RING
    for r in range(RING):
        g_wait(base + r, r)
        w_start(base + r, r)
    for r in range(RING):
        w_wait(base + r, r)


def _sc_gather(word_emb, idx2d):
    nchunks_total, chunk = idx2d.shape
    assert chunk == CHUNK and nchunks_total % NW == 0
    nchunks = nchunks_total // NW
    assert nchunks % RING == 0
    n = nchunks_total * CHUNK
    mesh = plsc.VectorSubcoreMesh(core_axis_name="c", subcore_axis_name="s")
    return pl.kernel(
        functools.partial(_sc_gather_body, nchunks),
        out_type=jax.ShapeDtypeStruct((n, DIM), jnp.float32),
        mesh=mesh,
        scratch_types=[
            pltpu.VMEM((nchunks, CHUNK), jnp.int32),
            pltpu.VMEM((RING, CHUNK, DIM), jnp.float32),
            pltpu.SemaphoreType.DMA((RING,)),
            pltpu.SemaphoreType.DMA((RING,)),
        ],
        compiler_params=pltpu.CompilerParams(use_tc_tiling_on_sc=False),
    )(idx2d, word_emb)


def _tc_ln_body(w_ref, seg_ref, pos_ref, tok_ref, gam_ref, bet_ref, o_ref):
    w = w_ref[...]
    seg = seg_ref[...]
    pos = pos_ref[...]
    tok0 = tok_ref[0]
    dtok = tok_ref[1] - tok0
    e = w + pos[None] + tok0[None, None, :] + seg.astype(jnp.float32) * dtok[None, None, :]
    ones = jnp.full((DIM, 1), 1.0 / DIM, dtype=jnp.float32)
    dn = (((2,), (0,)), ((), ()))
    m = lax.dot_general(e, ones, dn)
    q = lax.dot_general(e * e, ones, dn)
    var = q - m * m
    o_ref[...] = (e - m) * lax.rsqrt(var + EPS) * gam_ref[0][None, None, :] + bet_ref[0][None, None, :]


def _tc_ln(w, seg, pos, tok, gamma, beta):
    b, l, d = w.shape
    bb = 32
    grid = (b // bb,)
    return pl.pallas_call(
        _tc_ln_body,
        grid=grid,
        in_specs=[
            pl.BlockSpec((bb, l, d), lambda i: (i, 0, 0)),
            pl.BlockSpec((bb, l, 1), lambda i: (i, 0, 0)),
            pl.BlockSpec((l, d), lambda i: (0, 0)),
            pl.BlockSpec((2, d), lambda i: (0, 0)),
            pl.BlockSpec((1, d), lambda i: (0, 0)),
            pl.BlockSpec((1, d), lambda i: (0, 0)),
        ],
        out_specs=pl.BlockSpec((bb, l, d), lambda i: (i, 0, 0)),
        out_shape=jax.ShapeDtypeStruct((b, l, d), jnp.float32),
    )(w, seg, pos, tok, gamma, beta)


def kernel(x, seg, word_emb, pos_emb, tok_emb, gamma, beta):
    b, l = x.shape
    idx2d = x.reshape(-1, CHUNK).astype(jnp.int32)
    w = _sc_gather(word_emb, idx2d)
    w = w.reshape(b, l, DIM)
    return _tc_ln(
        w,
        seg.reshape(b, l, 1),
        pos_emb[:l],
        tok_emb,
        gamma.reshape(1, DIM),
        beta.reshape(1, DIM),
    )


# speedup vs baseline: 4.1538x; 1.2557x over previous
import functools

import jax
import jax.numpy as jnp
from jax import lax
from jax.experimental import pallas as pl
from jax.experimental.pallas import tpu as pltpu
from jax.experimental.pallas import tpu_sc as plsc

DIM = 64
EPS = 1e-12

NC = 2
NS = 16
NW = NC * NS

CHUNK = 128
RING = 8

BB = 3200
SB = BB // CHUNK


def _sc_gather_body(nchunks, idx_hbm, tab_hbm, out_hbm, idx_v, rows_v, gsem, wsem):
    wid = lax.axis_index("s") * NC + lax.axis_index("c")
    chunk0 = wid * nchunks

    pltpu.sync_copy(idx_hbm.at[pl.ds(chunk0, nchunks)], idx_v)

    def g_start(c, r):
        pltpu.async_copy(tab_hbm.at[idx_v.at[c]], rows_v.at[r], gsem.at[r])

    def g_wait(c, r):
        pltpu.make_async_copy(tab_hbm.at[idx_v.at[c]], rows_v.at[r], gsem.at[r]).wait()

    def w_start(c, r):
        pltpu.async_copy(rows_v.at[r], out_hbm.at[pl.ds((chunk0 + c) * CHUNK, CHUNK)], wsem.at[r])

    def w_wait(c, r):
        pltpu.make_async_copy(rows_v.at[r], out_hbm.at[pl.ds((chunk0 + c) * CHUNK, CHUNK)], wsem.at[r]).wait()

    ngroups = nchunks // RING
    for r in range(RING):
        g_start(r, r)

    def group(g, carry):
        base = g * RING
        for r in range(RING):
            g_wait(base + r, r)
            w_start(base + r, r)
        for r in range(RING):
            w_wait(base + r, r)
            g_start(base + RING + r, r)
        return carry

    lax.fori_loop(0, ngroups - 1, group, 0)

    base = (ngroups - 1) * RING
    for r in range(RING):
        g_wait(base + r, r)
        w_start(base + r, r)
    for r in range(RING):
        w_wait(base + r, r)


def _sc_gather(word_emb, idx2d):
    nchunks_total, chunk = idx2d.shape
    assert chunk == CHUNK and nchunks_total % NW == 0
    nchunks = nchunks_total // NW
    assert nchunks % RING == 0
    n = nchunks_total * CHUNK
    mesh = plsc.VectorSubcoreMesh(core_axis_name="c", subcore_axis_name="s")
    return pl.kernel(
        functools.partial(_sc_gather_body, nchunks),
        out_type=jax.ShapeDtypeStruct((n, DIM), jnp.float32),
        mesh=mesh,
        scratch_types=[
            pltpu.VMEM((nchunks, CHUNK), jnp.int32),
            pltpu.VMEM((RING, CHUNK, DIM), jnp.float32),
            pltpu.SemaphoreType.DMA((RING,)),
            pltpu.SemaphoreType.DMA((RING,)),
        ],
        compiler_params=pltpu.CompilerParams(use_tc_tiling_on_sc=False),
    )(idx2d, word_emb)


def _tc_ln_body(w_ref, se_ref, so_ref, pos_ref, emat_ref, oh_ref, tokc_ref, o_ref):
    w = w_ref[...]
    pos = pos_ref[...]
    emat = emat_ref[...]
    oh = oh_ref[...]
    tok0 = tokc_ref[0][None, :]
    dtok = tokc_ref[1][None, :]
    mlow = tokc_ref[2][None, :]
    gam = tokc_ref[3][None, :]
    bet = tokc_ref[4][None, :]

    dn2 = (((1,), (0,)), ((), ()))
    se = se_ref[0].astype(jnp.float32)
    so = so_ref[0].astype(jnp.float32)
    sev = jnp.sum(lax.dot_general(emat, se, dn2) * oh, axis=1, keepdims=True)
    sov = jnp.sum(lax.dot_general(emat, so, dn2) * oh, axis=1, keepdims=True)
    segx = sev * mlow + sov * (1.0 - mlow)

    e = w + pos + tok0 + segx * dtok

    hmat = jnp.concatenate([mlow, 1.0 - mlow], axis=0) * (1.0 / DIM)
    m2 = lax.dot_general(e, hmat.T, dn2)
    q2 = lax.dot_general(e * e, hmat.T, dn2)
    var2 = q2 - m2 * m2
    rs2 = lax.rsqrt(var2 + EPS)
    gmat = jnp.concatenate([mlow, 1.0 - mlow], axis=0)
    mexp = lax.dot_general(m2, gmat, dn2)
    rsexp = lax.dot_general(rs2, gmat, dn2)
    o_ref[...] = (e - mexp) * rsexp * gam + bet


def _tc_ln(w2, se3, so3, pos_t, emat, oh, tokc):
    nrows = w2.shape[0]
    grid = (nrows // BB,)
    return pl.pallas_call(
        _tc_ln_body,
        grid=grid,
        in_specs=[
            pl.BlockSpec((BB, 128), lambda i: (i, 0)),
            pl.BlockSpec((1, SB, 128), lambda i: (i, 0, 0)),
            pl.BlockSpec((1, SB, 128), lambda i: (i, 0, 0)),
            pl.BlockSpec((BB, 128), lambda i: (0, 0)),
            pl.BlockSpec((BB, SB), lambda i: (0, 0)),
            pl.BlockSpec((BB, 128), lambda i: (0, 0)),
            pl.BlockSpec((5, 128), lambda i: (0, 0)),
        ],
        out_specs=pl.BlockSpec((BB, 128), lambda i: (i, 0)),
        out_shape=jax.ShapeDtypeStruct((nrows, 128), jnp.float32),
    )(w2, se3, so3, pos_t, emat, oh, tokc)


def kernel(x, seg, word_emb, pos_emb, tok_emb, gamma, beta):
    b, l = x.shape
    n = b * l
    nrows = n // 2
    nblocks = nrows // BB

    idx2d = x.reshape(-1, CHUNK).astype(jnp.int32)
    w = _sc_gather(word_emb, idx2d)
    w2 = w.reshape(nrows, 128)

    seg_flat = seg.reshape(-1)
    se3 = seg_flat[0::2].reshape(nblocks, SB, 128)
    so3 = seg_flat[1::2].reshape(nblocks, SB, 128)

    pos128 = pos_emb[:l].reshape(l // 2, 128)
    pos_t = jnp.tile(pos128, (BB // (l // 2), 1))

    jrow = jnp.arange(BB, dtype=jnp.int32)
    emat = (jrow[:, None] // 128 == jnp.arange(SB, dtype=jnp.int32)[None, :]).astype(jnp.float32)
    oh = (jrow[:, None] % 128 == jnp.arange(128, dtype=jnp.int32)[None, :]).astype(jnp.float32)

    lane = jnp.arange(128, dtype=jnp.int32)
    mlow = (lane < DIM).astype(jnp.float32)
    tok0x = jnp.tile(tok_emb[0], 2)
    dtokx = jnp.tile(tok_emb[1] - tok_emb[0], 2)
    g128 = jnp.tile(gamma, 2)
    b128 = jnp.tile(beta, 2)
    tokc = jnp.stack([tok0x, dtokx, mlow, g128, b128], axis=0)

    out2 = _tc_ln(w2, se3, so3, pos_t, emat, oh, tokc)
    return out2.reshape(b, l, DIM)


# speedup vs baseline: 4.2840x; 1.0314x over previous
import functools

import jax
import jax.numpy as jnp
from jax import lax
from jax.experimental import pallas as pl
from jax.experimental.pallas import tpu as pltpu
from jax.experimental.pallas import tpu_sc as plsc

DIM = 64
EPS = 1e-12

NC = 2
NS = 16
NW = NC * NS

CHUNK = 128
RING = 8

BB = 3200
SB = 2 * BB // 128


def _sc_gather_body(nchunks, idx_hbm, tab_hbm, out_hbm, idx_v, rows_v, gsem, wsem):
    wid = lax.axis_index("s") * NC + lax.axis_index("c")
    chunk0 = wid * nchunks

    pltpu.sync_copy(idx_hbm.at[pl.ds(chunk0, nchunks)], idx_v)

    def g_start(c, r):
        pltpu.async_copy(tab_hbm.at[idx_v.at[c]], rows_v.at[r], gsem.at[r])

    def g_wait(c, r):
        pltpu.make_async_copy(tab_hbm.at[idx_v.at[c]], rows_v.at[r], gsem.at[r]).wait()

    def w_start(c, r):
        pltpu.async_copy(rows_v.at[r], out_hbm.at[pl.ds((chunk0 + c) * CHUNK, CHUNK)], wsem.at[r])

    def w_wait(c, r):
        pltpu.make_async_copy(rows_v.at[r], out_hbm.at[pl.ds((chunk0 + c) * CHUNK, CHUNK)], wsem.at[r]).wait()

    ngroups = nchunks // RING
    for r in range(RING):
        g_start(r, r)

    def group(g, carry):
        base = g * RING
        for r in range(RING):
            g_wait(base + r, r)
            w_start(base + r, r)
        for r in range(RING):
            w_wait(base + r, r)
            g_start(base + RING + r, r)
        return carry

    lax.fori_loop(0, ngroups - 1, group, 0)

    base = (ngroups - 1) * RING
    for r in range(RING):
        g_wait(base + r, r)
        w_start(base + r, r)
    for r in range(RING):
        w_wait(base + r, r)


def _sc_gather(word_emb, idx2d):
    nchunks_total, chunk = idx2d.shape
    assert chunk == CHUNK and nchunks_total % NW == 0
    nchunks = nchunks_total // NW
    assert nchunks % RING == 0
    n = nchunks_total * CHUNK
    mesh = plsc.VectorSubcoreMesh(core_axis_name="c", subcore_axis_name="s")
    return pl.kernel(
        functools.partial(_sc_gather_body, nchunks),
        out_type=jax.ShapeDtypeStruct((n, DIM), jnp.float32),
        mesh=mesh,
        scratch_types=[
            pltpu.VMEM((nchunks, CHUNK), jnp.int32),
            pltpu.VMEM((RING, CHUNK, DIM), jnp.float32),
            pltpu.SemaphoreType.DMA((RING,)),
            pltpu.SemaphoreType.DMA((RING,)),
        ],
        compiler_params=pltpu.CompilerParams(use_tc_tiling_on_sc=False),
    )(idx2d, word_emb)


def _tc_ln_body(w_ref, sg_ref, pos_ref, emat_ref, ohe_ref, oho_ref, tokc_ref, o_ref):
    w = w_ref[...]
    pos = pos_ref[...]
    emat = emat_ref[...]
    ohe = ohe_ref[...]
    oho = oho_ref[...]
    tok0 = tokc_ref[0][None, :]
    dtok = tokc_ref[1][None, :]
    mlow = tokc_ref[2][None, :]
    gam = tokc_ref[3][None, :]
    bet = tokc_ref[4][None, :]

    dn2 = (((1,), (0,)), ((), ()))
    sg = sg_ref[0].astype(jnp.float32)
    c = lax.dot_general(emat, sg, dn2)
    sev = jnp.sum(c * ohe, axis=1, keepdims=True)
    sov = jnp.sum(c * oho, axis=1, keepdims=True)
    segx = sev * mlow + sov * (1.0 - mlow)

    e = w + pos + tok0 + segx * dtok

    havg = jnp.concatenate([mlow, 1.0 - mlow], axis=0) * (1.0 / DIM)
    gmat = jnp.concatenate([mlow, 1.0 - mlow], axis=0)
    m2 = lax.dot_general(e, havg.T, dn2)
    mexp = lax.dot_general(m2, gmat, dn2)
    d = e - mexp
    q2 = lax.dot_general(d * d, havg.T, dn2)
    rs2 = lax.rsqrt(q2 + EPS)
    rsexp = lax.dot_general(rs2, gmat, dn2)
    o_ref[...] = d * rsexp * gam + bet


def _tc_ln(w2, sg3, pos_t, emat, ohe, oho, tokc):
    nrows = w2.shape[0]
    grid = (nrows // BB,)
    return pl.pallas_call(
        _tc_ln_body,
        grid=grid,
        in_specs=[
            pl.BlockSpec((BB, 128), lambda i: (i, 0)),
            pl.BlockSpec((1, SB, 128), lambda i: (i, 0, 0)),
            pl.BlockSpec((BB, 128), lambda i: (0, 0)),
            pl.BlockSpec((BB, SB), lambda i: (0, 0)),
            pl.BlockSpec((BB, 128), lambda i: (0, 0)),
            pl.BlockSpec((BB, 128), lambda i: (0, 0)),
            pl.BlockSpec((5, 128), lambda i: (0, 0)),
        ],
        out_specs=pl.BlockSpec((BB, 128), lambda i: (i, 0)),
        out_shape=jax.ShapeDtypeStruct((nrows, 128), jnp.float32),
    )(w2, sg3, pos_t, emat, ohe, oho, tokc)


def kernel(x, seg, word_emb, pos_emb, tok_emb, gamma, beta):
    b, l = x.shape
    n = b * l
    nrows = n // 2
    nblocks = nrows // BB

    idx2d = x.reshape(-1, CHUNK).astype(jnp.int32)
    w = _sc_gather(word_emb, idx2d)
    w2 = w.reshape(nrows, 128)

    sg3 = seg.reshape(nblocks, SB, 128)

    pos128 = pos_emb[:l].reshape(l // 2, 128)
    pos_t = jnp.tile(pos128, (BB // (l // 2), 1))

    jrow = jnp.arange(BB, dtype=jnp.int32)
    lane = jnp.arange(128, dtype=jnp.int32)
    emat = (jrow[:, None] // 64 == jnp.arange(SB, dtype=jnp.int32)[None, :]).astype(jnp.float32)
    ohe = (lane[None, :] == 2 * (jrow[:, None] % 64)).astype(jnp.float32)
    oho = (lane[None, :] == 2 * (jrow[:, None] % 64) + 1).astype(jnp.float32)

    mlow = (lane < DIM).astype(jnp.float32)
    tok0x = jnp.tile(tok_emb[0], 2)
    dtokx = jnp.tile(tok_emb[1] - tok_emb[0], 2)
    g128 = jnp.tile(gamma, 2)
    b128 = jnp.tile(beta, 2)
    tokc = jnp.stack([tok0x, dtokx, mlow, g128, b128], axis=0)

    out2 = _tc_ln(w2, sg3, pos_t, emat, ohe, oho, tokc)
    return out2.reshape(b, l, DIM)
